# baseline (speedup 1.0000x reference)
import jax
import jax.numpy as jnp
from jax import lax
from jax.experimental import pallas as pl
from jax.experimental.pallas import tpu as pltpu
from jax.experimental.pallas import tpu_sc as plsc

N_NODES = 10000
N_EDGES = 320000
D = 128

NC = 2
NS = 16
NW = NC * NS
L = 16

K = 64
NCHUNK = 160
EPW = K * NCHUNK
E_PAD = EPW * NW
N_PAD = 10240
DUMP_ROW = N_NODES
ROWS_PER_TILE = N_PAD // NS
DEGW = 16

_sc_mesh = plsc.VectorSubcoreMesh(core_axis_name="c", subcore_axis_name="s")


def _fill_f32(ref, rows, width, value):
    v = jnp.full((L,), value, jnp.float32)

    @pl.loop(0, rows)
    def _(r):
        for j in range(width // L):
            ref[r, pl.ds(j * L, L)] = v


def _deg_body(dst_hbm, deg_out_hbm, deg_acc, didx_v, ones_v, zbuf_v):
    cid = lax.axis_index("c")
    tid = lax.axis_index("s")
    wid = tid * NC + cid

    _fill_f32(ones_v, L, DEGW, 1.0)
    _fill_f32(zbuf_v, 64, DEGW, 0.0)
    for r in range(ROWS_PER_TILE // 64):
        pltpu.sync_copy(
            zbuf_v, deg_acc.at[pl.ds(tid * ROWS_PER_TILE + r * 64, 64)])
    pltpu.sync_copy(dst_hbm.at[pl.ds(wid * EPW, EPW)], didx_v)
    plsc.subcore_barrier()

    @pl.loop(0, EPW // L)
    def _(g):
        idx = didx_v[pl.ds(g * L, L)]
        pltpu.sync_copy(ones_v, deg_acc.at[idx], add=True)

    plsc.subcore_barrier()
    row0 = tid * ROWS_PER_TILE
    pltpu.sync_copy(
        deg_acc.at[pl.ds(row0, ROWS_PER_TILE)],
        deg_out_hbm.at[pl.ds(cid * N_PAD + row0, ROWS_PER_TILE)])


_deg_pass = pl.kernel(
    _deg_body,
    out_type=jax.ShapeDtypeStruct((NC * N_PAD, DEGW), jnp.float32),
    mesh=_sc_mesh,
    scratch_types=[
        pltpu.VMEM_SHARED((N_PAD, DEGW), jnp.float32),
        pltpu.VMEM((EPW,), jnp.int32),
        pltpu.VMEM((L, DEGW), jnp.float32),
        pltpu.VMEM((64, DEGW), jnp.float32),
    ],
)


def _seg_body(hp_hbm, src_hbm, dst_hbm, out_hbm,
              acc, sidx_v, didx_v, rows0_v, rows1_v,
              semg0, semg1, sems0, sems1):
    cid = lax.axis_index("c")
    tid = lax.axis_index("s")
    wid = tid * NC + cid
    rows = (rows0_v, rows1_v)
    semg = (semg0, semg1)
    sems = (sems0, sems1)

    _fill_f32(rows0_v, K, D, 0.0)
    for r in range(ROWS_PER_TILE // K):
        pltpu.sync_copy(
            rows0_v, acc.at[pl.ds(tid * ROWS_PER_TILE + r * K, K)])
    pltpu.sync_copy(src_hbm.at[pl.ds(wid * EPW, EPW)], sidx_v)
    pltpu.sync_copy(dst_hbm.at[pl.ds(wid * EPW, EPW)], didx_v)
    plsc.subcore_barrier()

    @pl.loop(0, NCHUNK, step=2)
    def _(c0):
        for b in range(2):
            c = c0 + b

            @pl.when(c >= 2)
            def _():
                for g in range(K // L):
                    idx = didx_v[pl.ds((c - 2) * K + g * L, L)]
                    pltpu.make_async_copy(
                        rows[b].at[pl.ds(g * L, L)], acc.at[idx],
                        sems[b]).wait()

            pltpu.async_copy(
                hp_hbm.at[sidx_v.at[pl.ds(c * K, K)]], rows[b],
                semg[b]).wait()
            for g in range(K // L):
                idx = didx_v[pl.ds(c * K + g * L, L)]
                pltpu.async_copy(
                    rows[b].at[pl.ds(g * L, L)], acc.at[idx], sems[b],
                    add=True)

    for b in range(2):
        c = NCHUNK - 2 + b
        for g in range(K // L):
            idx = didx_v[pl.ds(c * K + g * L, L)]
            pltpu.make_async_copy(
                rows[b].at[pl.ds(g * L, L)], acc.at[idx], sems[b]).wait()
    plsc.subcore_barrier()
    row0 = tid * ROWS_PER_TILE
    pltpu.sync_copy(
        acc.at[pl.ds(row0, ROWS_PER_TILE)],
        out_hbm.at[pl.ds(cid * N_PAD + row0, ROWS_PER_TILE)])


_seg_pass = pl.kernel(
    _seg_body,
    out_type=jax.ShapeDtypeStruct((NC * N_PAD, D), jnp.float32),
    mesh=_sc_mesh,
    scratch_types=[
        pltpu.VMEM_SHARED((N_PAD, D), jnp.float32),
        pltpu.VMEM((EPW,), jnp.int32),
        pltpu.VMEM((EPW,), jnp.int32),
        pltpu.VMEM((K, D), jnp.float32),
        pltpu.VMEM((K, D), jnp.float32),
        pltpu.SemaphoreType.DMA,
        pltpu.SemaphoreType.DMA,
        pltpu.SemaphoreType.DMA,
        pltpu.SemaphoreType.DMA,
    ],
)


R = 1000
G = N_NODES // R


def _mm1_body(degp_ref, x_ref, w_ref, hp_ref, dinv_ref):
    deg = degp_ref[0, :, 0:1] + degp_ref[1, :, 0:1] + 1.0
    dinv = lax.rsqrt(deg)
    hp = jnp.dot(x_ref[...], w_ref[...],
                 preferred_element_type=jnp.float32) * dinv
    hp_ref[...] = hp
    dinv_ref[...] = dinv


_mm1 = pl.pallas_call(
    _mm1_body,
    grid=(G,),
    in_specs=[
        pl.BlockSpec((NC, R, DEGW), lambda i: (0, i, 0)),
        pl.BlockSpec((R, D), lambda i: (i, 0)),
        pl.BlockSpec((D, D), lambda i: (0, 0)),
    ],
    out_specs=[
        pl.BlockSpec((R, D), lambda i: (i, 0)),
        pl.BlockSpec((R, 1), lambda i: (i, 0)),
    ],
    out_shape=[
        jax.ShapeDtypeStruct((N_NODES, D), jnp.float32),
        jax.ShapeDtypeStruct((N_NODES, 1), jnp.float32),
    ],
)


def _mid_body(sp_ref, hp_ref, dinv_ref, b_ref, w_ref, out_ref):
    dinv = dinv_ref[...]
    h = jnp.maximum(
        dinv * (sp_ref[0] + sp_ref[1] + hp_ref[...]) + b_ref[...], 0.0)
    out_ref[...] = jnp.dot(
        h, w_ref[...], preferred_element_type=jnp.float32) * dinv


_mid = pl.pallas_call(
    _mid_body,
    grid=(G,),
    in_specs=[
        pl.BlockSpec((NC, R, D), lambda i: (0, i, 0)),
        pl.BlockSpec((R, D), lambda i: (i, 0)),
        pl.BlockSpec((R, 1), lambda i: (i, 0)),
        pl.BlockSpec((1, D), lambda i: (0, 0)),
        pl.BlockSpec((D, D), lambda i: (0, 0)),
    ],
    out_specs=pl.BlockSpec((R, D), lambda i: (i, 0)),
    out_shape=jax.ShapeDtypeStruct((N_NODES, D), jnp.float32),
)


def _head_body(sp_ref, hp_ref, dinv_ref, b_ref, w_ref, bo_ref, out_ref):
    h = jnp.maximum(
        dinv_ref[...] * (sp_ref[0] + sp_ref[1] + hp_ref[...]) + b_ref[...],
        0.0)
    out_ref[...] = jnp.dot(
        h, w_ref[...], preferred_element_type=jnp.float32) + bo_ref[...]


_head = pl.pallas_call(
    _head_body,
    grid=(G,),
    in_specs=[
        pl.BlockSpec((NC, R, D), lambda i: (0, i, 0)),
        pl.BlockSpec((R, D), lambda i: (i, 0)),
        pl.BlockSpec((R, 1), lambda i: (i, 0)),
        pl.BlockSpec((1, D), lambda i: (0, 0)),
        pl.BlockSpec((D, D), lambda i: (0, 0)),
        pl.BlockSpec((1, D), lambda i: (0, 0)),
    ],
    out_specs=pl.BlockSpec((R, D), lambda i: (i, 0)),
    out_shape=jax.ShapeDtypeStruct((N_NODES, D), jnp.float32),
)


def kernel(X, A, W1, b1, W2, b2, Wo, bo):
    n_classes = Wo.shape[1]
    pad = E_PAD - N_EDGES
    src = jnp.concatenate([A[0], jnp.zeros((pad,), A.dtype)])
    dst = jnp.concatenate([A[1], jnp.full((pad,), DUMP_ROW, A.dtype)])

    degp = _deg_pass(dst).reshape(NC, N_PAD, DEGW)
    hp1, dinv = _mm1(degp, X, W1)
    s1 = _seg_pass(hp1, src, dst).reshape(NC, N_PAD, D)
    hp2 = _mid(s1, hp1, dinv, b1.reshape(1, D), W2)
    s2 = _seg_pass(hp2, src, dst).reshape(NC, N_PAD, D)
    wo_p = jnp.pad(Wo, ((0, 0), (0, D - n_classes)))
    bo_p = jnp.pad(bo, (0, D - n_classes)).reshape(1, D)
    out = _head(s2, hp2, dinv, b2.reshape(1, D), wo_p, bo_p)
    return out[:, :n_classes]


# speedup vs baseline: 1.1407x; 1.1407x over previous
import jax
import jax.numpy as jnp
from jax import lax
from jax.experimental import pallas as pl
from jax.experimental.pallas import tpu as pltpu
from jax.experimental.pallas import tpu_sc as plsc

N_NODES = 10000
N_EDGES = 320000
D = 128

NC = 2
NS = 16
NW = NC * NS
L = 16

K = 64
NCHUNK = 160
EPW = K * NCHUNK
E_PAD = EPW * NW
PAIRW = 2 * EPW
EPW0 = 14336
EPW1 = PAIRW - EPW0
N_PAD = 10240
DUMP_ROW = N_NODES
ROWS_PER_TILE = N_PAD // NS
DEGW = 16

_sc_mesh = plsc.VectorSubcoreMesh(core_axis_name="c", subcore_axis_name="s")


def _fill_f32(ref, rows, width, value):
    v = jnp.full((L,), value, jnp.float32)

    @pl.loop(0, rows)
    def _(r):
        for j in range(width // L):
            ref[r, pl.ds(j * L, L)] = v


def _deg_body(dst_hbm, deg_out_hbm, deg_acc, didx_v, ones_v, zbuf_v):
    cid = lax.axis_index("c")
    tid = lax.axis_index("s")
    wid = tid * NC + cid

    _fill_f32(ones_v, L, DEGW, 1.0)
    _fill_f32(zbuf_v, 64, DEGW, 0.0)
    for r in range(ROWS_PER_TILE // 64):
        pltpu.sync_copy(
            zbuf_v, deg_acc.at[pl.ds(tid * ROWS_PER_TILE + r * 64, 64)])
    pltpu.sync_copy(dst_hbm.at[pl.ds(wid * EPW, EPW)], didx_v)
    plsc.subcore_barrier()

    @pl.loop(0, EPW // L)
    def _(g):
        idx = didx_v[pl.ds(g * L, L)]
        pltpu.sync_copy(ones_v, deg_acc.at[idx], add=True)

    plsc.subcore_barrier()
    row0 = tid * ROWS_PER_TILE
    pltpu.sync_copy(
        deg_acc.at[pl.ds(row0, ROWS_PER_TILE)],
        deg_out_hbm.at[pl.ds(cid * N_PAD + row0, ROWS_PER_TILE)])


_deg_pass = pl.kernel(
    _deg_body,
    out_type=jax.ShapeDtypeStruct((NC * N_PAD, DEGW), jnp.float32),
    mesh=_sc_mesh,
    scratch_types=[
        pltpu.VMEM_SHARED((N_PAD, DEGW), jnp.float32),
        pltpu.VMEM((EPW,), jnp.int32),
        pltpu.VMEM((L, DEGW), jnp.float32),
        pltpu.VMEM((64, DEGW), jnp.float32),
    ],
)


def _seg_body(hp_hbm, src_hbm, dst_hbm, out_hbm,
              acc, sidx_v, didx_v, rows0_v, rows1_v,
              semg0, semg1, sems0, sems1):
    cid = lax.axis_index("c")
    tid = lax.axis_index("s")
    rows = (rows0_v, rows1_v)
    semg = (semg0, semg1)
    sems = (sems0, sems1)

    nchunk = jnp.where(cid == 0, EPW0 // K, EPW1 // K)

    _fill_f32(rows0_v, K, D, 0.0)
    for r in range(ROWS_PER_TILE // K):
        pltpu.sync_copy(
            rows0_v, acc.at[pl.ds(tid * ROWS_PER_TILE + r * K, K)])

    @pl.when(cid == 0)
    def _():
        pltpu.sync_copy(src_hbm.at[pl.ds(tid * PAIRW, EPW0)],
                        sidx_v.at[pl.ds(0, EPW0)])
        pltpu.sync_copy(dst_hbm.at[pl.ds(tid * PAIRW, EPW0)],
                        didx_v.at[pl.ds(0, EPW0)])

    @pl.when(cid == 1)
    def _():
        pltpu.sync_copy(src_hbm.at[pl.ds(tid * PAIRW + EPW0, EPW1)],
                        sidx_v.at[pl.ds(0, EPW1)])
        pltpu.sync_copy(dst_hbm.at[pl.ds(tid * PAIRW + EPW0, EPW1)],
                        didx_v.at[pl.ds(0, EPW1)])

    plsc.subcore_barrier()

    @pl.loop(0, nchunk, step=2)
    def _(c0):
        for b in range(2):
            c = c0 + b

            @pl.when(c >= 2)
            def _():
                for g in range(K // L):
                    idx = didx_v[pl.ds((c - 2) * K + g * L, L)]
                    pltpu.make_async_copy(
                        rows[b].at[pl.ds(g * L, L)], acc.at[idx],
                        sems[b]).wait()

            pltpu.async_copy(
                hp_hbm.at[sidx_v.at[pl.ds(c * K, K)]], rows[b],
                semg[b]).wait()
            for g in range(K // L):
                idx = didx_v[pl.ds(c * K + g * L, L)]
                pltpu.async_copy(
                    rows[b].at[pl.ds(g * L, L)], acc.at[idx], sems[b],
                    add=True)

    for b in range(2):
        c = nchunk - 2 + b
        for g in range(K // L):
            idx = didx_v[pl.ds(c * K + g * L, L)]
            pltpu.make_async_copy(
                rows[b].at[pl.ds(g * L, L)], acc.at[idx], sems[b]).wait()
    plsc.subcore_barrier()
    row0 = tid * ROWS_PER_TILE
    pltpu.sync_copy(
        acc.at[pl.ds(row0, ROWS_PER_TILE)],
        out_hbm.at[pl.ds(cid * N_PAD + row0, ROWS_PER_TILE)])


_seg_pass = pl.kernel(
    _seg_body,
    out_type=jax.ShapeDtypeStruct((NC * N_PAD, D), jnp.float32),
    mesh=_sc_mesh,
    scratch_types=[
        pltpu.VMEM_SHARED((N_PAD, D), jnp.float32),
        pltpu.VMEM((EPW0,), jnp.int32),
        pltpu.VMEM((EPW0,), jnp.int32),
        pltpu.VMEM((K, D), jnp.float32),
        pltpu.VMEM((K, D), jnp.float32),
        pltpu.SemaphoreType.DMA,
        pltpu.SemaphoreType.DMA,
        pltpu.SemaphoreType.DMA,
        pltpu.SemaphoreType.DMA,
    ],
)


R = 1000
G = N_NODES // R


def _mm1_body(degp_ref, x_ref, w_ref, hp_ref, dinv_ref):
    deg = degp_ref[0, :, 0:1] + degp_ref[1, :, 0:1] + 1.0
    dinv = lax.rsqrt(deg)
    hp = jnp.dot(x_ref[...], w_ref[...],
                 preferred_element_type=jnp.float32) * dinv
    hp_ref[...] = hp
    dinv_ref[...] = dinv


_mm1 = pl.pallas_call(
    _mm1_body,
    grid=(G,),
    in_specs=[
        pl.BlockSpec((NC, R, DEGW), lambda i: (0, i, 0)),
        pl.BlockSpec((R, D), lambda i: (i, 0)),
        pl.BlockSpec((D, D), lambda i: (0, 0)),
    ],
    out_specs=[
        pl.BlockSpec((R, D), lambda i: (i, 0)),
        pl.BlockSpec((R, 1), lambda i: (i, 0)),
    ],
    out_shape=[
        jax.ShapeDtypeStruct((N_NODES, D), jnp.float32),
        jax.ShapeDtypeStruct((N_NODES, 1), jnp.float32),
    ],
)


def _mid_body(sp_ref, hp_ref, dinv_ref, b_ref, w_ref, out_ref):
    dinv = dinv_ref[...]
    h = jnp.maximum(
        dinv * (sp_ref[0] + sp_ref[1] + hp_ref[...]) + b_ref[...], 0.0)
    out_ref[...] = jnp.dot(
        h, w_ref[...], preferred_element_type=jnp.float32) * dinv


_mid = pl.pallas_call(
    _mid_body,
    grid=(G,),
    in_specs=[
        pl.BlockSpec((NC, R, D), lambda i: (0, i, 0)),
        pl.BlockSpec((R, D), lambda i: (i, 0)),
        pl.BlockSpec((R, 1), lambda i: (i, 0)),
        pl.BlockSpec((1, D), lambda i: (0, 0)),
        pl.BlockSpec((D, D), lambda i: (0, 0)),
    ],
    out_specs=pl.BlockSpec((R, D), lambda i: (i, 0)),
    out_shape=jax.ShapeDtypeStruct((N_NODES, D), jnp.float32),
)


def _head_body(sp_ref, hp_ref, dinv_ref, b_ref, w_ref, bo_ref, out_ref):
    h = jnp.maximum(
        dinv_ref[...] * (sp_ref[0] + sp_ref[1] + hp_ref[...]) + b_ref[...],
        0.0)
    out_ref[...] = jnp.dot(
        h, w_ref[...], preferred_element_type=jnp.float32) + bo_ref[...]


_head = pl.pallas_call(
    _head_body,
    grid=(G,),
    in_specs=[
        pl.BlockSpec((NC, R, D), lambda i: (0, i, 0)),
        pl.BlockSpec((R, D), lambda i: (i, 0)),
        pl.BlockSpec((R, 1), lambda i: (i, 0)),
        pl.BlockSpec((1, D), lambda i: (0, 0)),
        pl.BlockSpec((D, D), lambda i: (0, 0)),
        pl.BlockSpec((1, D), lambda i: (0, 0)),
    ],
    out_specs=pl.BlockSpec((R, D), lambda i: (i, 0)),
    out_shape=jax.ShapeDtypeStruct((N_NODES, D), jnp.float32),
)


def kernel(X, A, W1, b1, W2, b2, Wo, bo):
    n_classes = Wo.shape[1]
    pad = E_PAD - N_EDGES
    src = jnp.concatenate([A[0], jnp.zeros((pad,), A.dtype)])
    dst = jnp.concatenate([A[1], jnp.full((pad,), DUMP_ROW, A.dtype)])

    degp = _deg_pass(dst).reshape(NC, N_PAD, DEGW)
    hp1, dinv = _mm1(degp, X, W1)
    s1 = _seg_pass(hp1, src, dst).reshape(NC, N_PAD, D)
    hp2 = _mid(s1, hp1, dinv, b1.reshape(1, D), W2)
    s2 = _seg_pass(hp2, src, dst).reshape(NC, N_PAD, D)
    wo_p = jnp.pad(Wo, ((0, 0), (0, D - n_classes)))
    bo_p = jnp.pad(bo, (0, D - n_classes)).reshape(1, D)
    out = _head(s2, hp2, dinv, b2.reshape(1, D), wo_p, bo_p)
    return out[:, :n_classes]
